# baseline (speedup 1.0000x reference)
import functools

import jax
import jax.numpy as jnp
from jax import lax
from jax.experimental import pallas as pl
from jax.experimental.pallas import tpu as pltpu
from jax.experimental.pallas import tpu_sc as plsc

HEADS = 16
LANES = 16
NUM_WORKERS = 32
ROWS_PER_CHUNK = 2
UNROLL = 8

_GATHER_DNUMS = lax.GatherDimensionNumbers(
    offset_dims=(), collapsed_slice_dims=(0,), start_index_map=(0,))


def _vreg_gather(vec, lane_ids):
    return lax.gather(vec, lane_ids.reshape(LANES, 1), _GATHER_DNUMS, (1,),
                      mode=lax.GatherScatterMode.PROMISE_IN_BOUNDS)


def _build_sc_kernel(n_rows: int, n_cols: int):
    rows_w = n_rows // NUM_WORKERS
    idx_w = rows_w * n_cols
    blk_w = n_cols * HEADS
    chunk_w = ROWS_PER_CHUNK * blk_w
    n_chunks = rows_w // ROWS_PER_CHUNK
    n_pairs = n_chunks // 2
    groups = chunk_w // (HEADS * LANES)
    n_total = n_rows * n_cols
    mesh = plsc.VectorSubcoreMesh(core_axis_name="c", subcore_axis_name="s")

    @functools.partial(
        pl.kernel,
        mesh=mesh,
        out_type=jax.ShapeDtypeStruct((n_total * HEADS,), jnp.float32),
        compiler_params=pltpu.CompilerParams(needs_layout_passes=False),
        scratch_types=(
            [pltpu.VMEM((13 * HEADS,), jnp.float32)]
            + [pltpu.VMEM((idx_w,), jnp.int32)]
            + [pltpu.VMEM((chunk_w,), jnp.float32) for _ in range(2)]
            + [pltpu.SemaphoreType.DMA] * 4
        ),
    )
    def sc_kernel(x_hbm, w_hbm, out_hbm, *scratch):
        w_vmem, idx_vmem, row0, row1 = scratch[:4]
        w_sem, idx_sem, out_sem0, out_sem1 = scratch[4:]
        row_bufs = (row0, row1)
        out_sems = (out_sem0, out_sem1)

        wid = lax.axis_index("s") * 2 + lax.axis_index("c")
        idx_base = wid * idx_w
        out_base = wid * rows_w * blk_w
        pltpu.async_copy(w_hbm, w_vmem, w_sem).wait()
        pltpu.async_copy(x_hbm.at[pl.ds(idx_base, idx_w)], idx_vmem,
                         idx_sem).wait()

        lane_iota = lax.iota(jnp.int32, LANES)
        w_cols = [plsc.load_gather(w_vmem, [lane_iota * HEADS + h])
                  for h in range(HEADS)]

        def compute_chunk(c, rb):
            @plsc.parallel_loop(0, groups, unroll=UNROLL)
            def body(u):
                il = c * ROWS_PER_CHUNK + (u >> 5)
                jpart = ((u >> 3) & 3) * 1024 + (u & 7) * LANES
                idx_vec = idx_vmem[pl.ds((il >> 3) * 4096 + (il & 7) * 128
                                         + jpart, LANES)]
                sbase = (u >> 5) * blk_w + jpart
                for h in range(HEADS):
                    val = _vreg_gather(w_cols[h], idx_vec)
                    off = (h // 8) * 4096 + (h % 8) * 128
                    row_bufs[rb][pl.ds(sbase + off, LANES)] = val

        def issue_out(c, rb):
            return pltpu.async_copy(
                row_bufs[rb],
                out_hbm.at[pl.ds(out_base + c * chunk_w, chunk_w)],
                out_sems[rb])

        def wait_out(rb):
            pltpu.make_async_copy(
                row_bufs[rb],
                out_hbm.at[pl.ds(0, chunk_w)],
                out_sems[rb]).wait()

        compute_chunk(0, 0)
        issue_out(0, 0)
        compute_chunk(1, 1)
        issue_out(1, 1)

        def pair_body(t, carry):
            c0 = t * 2
            wait_out(0)
            compute_chunk(c0, 0)
            issue_out(c0, 0)
            wait_out(1)
            compute_chunk(c0 + 1, 1)
            issue_out(c0 + 1, 1)
            return carry

        lax.fori_loop(1, n_pairs, pair_body, 0, unroll=False)

        wait_out(0)
        wait_out(1)

    return sc_kernel


def kernel(x, spatial_encoding_weight):
    rows, cols = x.shape
    x_flat = (x.reshape(rows // 8, 8, cols // 128, 128)
              .transpose(0, 2, 1, 3).reshape(-1))
    flat = _build_sc_kernel(rows, cols)(
        x_flat, spatial_encoding_weight.reshape(-1))
    f5 = flat.reshape(rows, HEADS // 8, cols // 128, 8, 128)
    return f5.transpose(0, 2, 4, 1, 3).reshape(rows, cols, HEADS)


# speedup vs baseline: 1.0802x; 1.0802x over previous
import functools

import jax
import jax.numpy as jnp
from jax import lax
from jax.experimental import pallas as pl
from jax.experimental.pallas import tpu as pltpu
from jax.experimental.pallas import tpu_sc as plsc

HEADS = 16
LANES = 16
NUM_WORKERS = 32
ROWS_PER_CHUNK = 2
UNROLL = 2

_GATHER_DNUMS = lax.GatherDimensionNumbers(
    offset_dims=(), collapsed_slice_dims=(0,), start_index_map=(0,))


def _vreg_gather(vec, lane_ids):
    return lax.gather(vec, lane_ids.reshape(LANES, 1), _GATHER_DNUMS, (1,),
                      mode=lax.GatherScatterMode.PROMISE_IN_BOUNDS)


def _build_sc_kernel(n_rows: int, n_cols: int):
    rows_w = n_rows // NUM_WORKERS
    idx_w = rows_w * n_cols
    blk_w = n_cols * HEADS
    chunk_w = ROWS_PER_CHUNK * blk_w
    n_chunks = rows_w // ROWS_PER_CHUNK
    n_pairs = n_chunks // 2
    groups = chunk_w // (HEADS * LANES)
    n_total = n_rows * n_cols
    mesh = plsc.VectorSubcoreMesh(core_axis_name="c", subcore_axis_name="s")

    @functools.partial(
        pl.kernel,
        mesh=mesh,
        out_type=jax.ShapeDtypeStruct((n_total * HEADS,), jnp.float32),
        compiler_params=pltpu.CompilerParams(needs_layout_passes=False),
        scratch_types=(
            [pltpu.VMEM((13 * HEADS,), jnp.float32)]
            + [pltpu.VMEM((idx_w,), jnp.int32)]
            + [pltpu.VMEM((chunk_w,), jnp.float32) for _ in range(2)]
            + [pltpu.SemaphoreType.DMA] * 4
        ),
    )
    def sc_kernel(x_hbm, w_hbm, out_hbm, *scratch):
        w_vmem, idx_vmem, row0, row1 = scratch[:4]
        w_sem, idx_sem, out_sem0, out_sem1 = scratch[4:]
        row_bufs = (row0, row1)
        out_sems = (out_sem0, out_sem1)

        wid = lax.axis_index("s") * 2 + lax.axis_index("c")
        idx_base = wid * idx_w
        out_base = wid * rows_w * blk_w
        pltpu.async_copy(w_hbm, w_vmem, w_sem).wait()
        pltpu.async_copy(x_hbm.at[pl.ds(idx_base, idx_w)], idx_vmem,
                         idx_sem).wait()

        lane_iota = lax.iota(jnp.int32, LANES)
        w_cols = [plsc.load_gather(w_vmem, [lane_iota * HEADS + h])
                  for h in range(HEADS)]

        def compute_chunk(c, rb):
            @plsc.parallel_loop(0, groups, unroll=UNROLL)
            def body(u):
                il = c * ROWS_PER_CHUNK + (u >> 5)
                jpart = ((u >> 3) & 3) * 1024 + (u & 7) * LANES
                idx_vec = idx_vmem[pl.ds((il >> 3) * 4096 + (il & 7) * 128
                                         + jpart, LANES)]
                sbase = (u >> 5) * blk_w + jpart
                for h in range(HEADS):
                    val = _vreg_gather(w_cols[h], idx_vec)
                    off = (h // 8) * 4096 + (h % 8) * 128
                    row_bufs[rb][pl.ds(sbase + off, LANES)] = val

        def issue_out(c, rb):
            return pltpu.async_copy(
                row_bufs[rb],
                out_hbm.at[pl.ds(out_base + c * chunk_w, chunk_w)],
                out_sems[rb])

        def wait_out(rb):
            pltpu.make_async_copy(
                row_bufs[rb],
                out_hbm.at[pl.ds(0, chunk_w)],
                out_sems[rb]).wait()

        compute_chunk(0, 0)
        issue_out(0, 0)
        compute_chunk(1, 1)
        issue_out(1, 1)

        def pair_body(t, carry):
            c0 = t * 2
            wait_out(0)
            compute_chunk(c0, 0)
            issue_out(c0, 0)
            wait_out(1)
            compute_chunk(c0 + 1, 1)
            issue_out(c0 + 1, 1)
            return carry

        lax.fori_loop(1, n_pairs, pair_body, 0, unroll=False)

        wait_out(0)
        wait_out(1)

    return sc_kernel


def kernel(x, spatial_encoding_weight):
    rows, cols = x.shape
    x_flat = (x.reshape(rows // 8, 8, cols // 128, 128)
              .transpose(0, 2, 1, 3).reshape(-1))
    flat = _build_sc_kernel(rows, cols)(
        x_flat, spatial_encoding_weight.reshape(-1))
    f5 = flat.reshape(rows, HEADS // 8, cols // 128, 8, 128)
    return f5.transpose(0, 2, 4, 1, 3).reshape(rows, cols, HEADS)


# speedup vs baseline: 1.0860x; 1.0054x over previous
import functools

import jax
import jax.numpy as jnp
from jax import lax
from jax.experimental import pallas as pl
from jax.experimental.pallas import tpu as pltpu
from jax.experimental.pallas import tpu_sc as plsc

HEADS = 16
LANES = 16
NUM_WORKERS = 32
ROWS_PER_CHUNK = 2
UNROLL = 1

_GATHER_DNUMS = lax.GatherDimensionNumbers(
    offset_dims=(), collapsed_slice_dims=(0,), start_index_map=(0,))


def _vreg_gather(vec, lane_ids):
    return lax.gather(vec, lane_ids.reshape(LANES, 1), _GATHER_DNUMS, (1,),
                      mode=lax.GatherScatterMode.PROMISE_IN_BOUNDS)


def _build_sc_kernel(n_rows: int, n_cols: int):
    rows_w = n_rows // NUM_WORKERS
    idx_w = rows_w * n_cols
    blk_w = n_cols * HEADS
    chunk_w = ROWS_PER_CHUNK * blk_w
    n_chunks = rows_w // ROWS_PER_CHUNK
    n_pairs = n_chunks // 2
    groups = chunk_w // (HEADS * LANES)
    n_total = n_rows * n_cols
    mesh = plsc.VectorSubcoreMesh(core_axis_name="c", subcore_axis_name="s")

    @functools.partial(
        pl.kernel,
        mesh=mesh,
        out_type=jax.ShapeDtypeStruct((n_total * HEADS,), jnp.float32),
        compiler_params=pltpu.CompilerParams(needs_layout_passes=False),
        scratch_types=(
            [pltpu.VMEM((13 * HEADS,), jnp.float32)]
            + [pltpu.VMEM((idx_w,), jnp.int32)]
            + [pltpu.VMEM((chunk_w,), jnp.float32) for _ in range(2)]
            + [pltpu.SemaphoreType.DMA] * 4
        ),
    )
    def sc_kernel(x_hbm, w_hbm, out_hbm, *scratch):
        w_vmem, idx_vmem, row0, row1 = scratch[:4]
        w_sem, idx_sem, out_sem0, out_sem1 = scratch[4:]
        row_bufs = (row0, row1)
        out_sems = (out_sem0, out_sem1)

        wid = lax.axis_index("s") * 2 + lax.axis_index("c")
        idx_base = wid * idx_w
        out_base = wid * rows_w * blk_w
        pltpu.async_copy(w_hbm, w_vmem, w_sem).wait()
        pltpu.async_copy(x_hbm.at[pl.ds(idx_base, idx_w)], idx_vmem,
                         idx_sem).wait()

        lane_iota = lax.iota(jnp.int32, LANES)
        w_cols = [plsc.load_gather(w_vmem, [lane_iota * HEADS + h])
                  for h in range(HEADS)]

        def compute_chunk(c, rb):
            @plsc.parallel_loop(0, groups, unroll=UNROLL)
            def body(u):
                il = c * ROWS_PER_CHUNK + (u >> 5)
                jpart = ((u >> 3) & 3) * 1024 + (u & 7) * LANES
                idx_vec = idx_vmem[pl.ds((il >> 3) * 4096 + (il & 7) * 128
                                         + jpart, LANES)]
                sbase = (u >> 5) * blk_w + jpart
                for h in range(HEADS):
                    val = _vreg_gather(w_cols[h], idx_vec)
                    off = (h // 8) * 4096 + (h % 8) * 128
                    row_bufs[rb][pl.ds(sbase + off, LANES)] = val

        def issue_out(c, rb):
            return pltpu.async_copy(
                row_bufs[rb],
                out_hbm.at[pl.ds(out_base + c * chunk_w, chunk_w)],
                out_sems[rb])

        def wait_out(rb):
            pltpu.make_async_copy(
                row_bufs[rb],
                out_hbm.at[pl.ds(0, chunk_w)],
                out_sems[rb]).wait()

        compute_chunk(0, 0)
        issue_out(0, 0)
        compute_chunk(1, 1)
        issue_out(1, 1)

        def pair_body(t, carry):
            c0 = t * 2
            wait_out(0)
            compute_chunk(c0, 0)
            issue_out(c0, 0)
            wait_out(1)
            compute_chunk(c0 + 1, 1)
            issue_out(c0 + 1, 1)
            return carry

        lax.fori_loop(1, n_pairs, pair_body, 0, unroll=False)

        wait_out(0)
        wait_out(1)

    return sc_kernel


def kernel(x, spatial_encoding_weight):
    rows, cols = x.shape
    x_flat = (x.reshape(rows // 8, 8, cols // 128, 128)
              .transpose(0, 2, 1, 3).reshape(-1))
    flat = _build_sc_kernel(rows, cols)(
        x_flat, spatial_encoding_weight.reshape(-1))
    f5 = flat.reshape(rows, HEADS // 8, cols // 128, 8, 128)
    return f5.transpose(0, 2, 4, 1, 3).reshape(rows, cols, HEADS)


# speedup vs baseline: 1.0959x; 1.0091x over previous
import functools

import jax
import jax.numpy as jnp
from jax import lax
from jax.experimental import pallas as pl
from jax.experimental.pallas import tpu as pltpu
from jax.experimental.pallas import tpu_sc as plsc

HEADS = 16
LANES = 16
NUM_WORKERS = 32
ROWS_PER_CHUNK = 2
UNROLL = 1

_GATHER_DNUMS = lax.GatherDimensionNumbers(
    offset_dims=(), collapsed_slice_dims=(0,), start_index_map=(0,))


def _vreg_gather(vec, lane_ids):
    return lax.gather(vec, lane_ids.reshape(LANES, 1), _GATHER_DNUMS, (1,),
                      mode=lax.GatherScatterMode.PROMISE_IN_BOUNDS)


def _build_sc_kernel(n_rows: int, n_cols: int):
    rows_w = n_rows // NUM_WORKERS
    idx_w = rows_w * n_cols
    blk_w = n_cols * HEADS
    chunk_w = ROWS_PER_CHUNK * blk_w
    n_chunks = rows_w // ROWS_PER_CHUNK
    n_pairs = n_chunks // 2
    groups = chunk_w // (HEADS * LANES)
    n_total = n_rows * n_cols
    mesh = plsc.VectorSubcoreMesh(core_axis_name="c", subcore_axis_name="s")

    @functools.partial(
        pl.kernel,
        mesh=mesh,
        out_type=jax.ShapeDtypeStruct((n_total * HEADS,), jnp.float32),
        compiler_params=pltpu.CompilerParams(needs_layout_passes=False),
        scratch_types=(
            [pltpu.VMEM((13 * HEADS,), jnp.float32)]
            + [pltpu.VMEM((idx_w,), jnp.int32)]
            + [pltpu.VMEM((chunk_w,), jnp.float32) for _ in range(2)]
            + [pltpu.SemaphoreType.DMA] * 5
        ),
    )
    def sc_kernel(x_hbm, w_hbm, out_hbm, *scratch):
        w_vmem, idx_vmem, row0, row1 = scratch[:4]
        w_sem, idx_sem, idx_sem2, out_sem0, out_sem1 = scratch[4:]
        row_bufs = (row0, row1)
        out_sems = (out_sem0, out_sem1)

        wid = lax.axis_index("s") * 2 + lax.axis_index("c")
        idx_base = wid * idx_w
        out_base = wid * rows_w * blk_w
        half_w = idx_w // 2
        w_handle = pltpu.async_copy(w_hbm, w_vmem, w_sem)
        h0 = pltpu.async_copy(
            x_hbm.at[pl.ds(idx_base, half_w)],
            idx_vmem.at[pl.ds(0, half_w)], idx_sem)
        pltpu.async_copy(
            x_hbm.at[pl.ds(idx_base + half_w, half_w)],
            idx_vmem.at[pl.ds(half_w, half_w)], idx_sem2)
        w_handle.wait()

        lane_iota = lax.iota(jnp.int32, LANES)
        w_cols = [plsc.load_gather(w_vmem, [lane_iota * HEADS + h])
                  for h in range(HEADS)]
        h0.wait()

        def compute_chunk(c, rb):
            @plsc.parallel_loop(0, groups, unroll=UNROLL)
            def body(u):
                il = c * ROWS_PER_CHUNK + (u >> 5)
                jpart = ((u >> 3) & 3) * 1024 + (u & 7) * LANES
                idx_vec = idx_vmem[pl.ds((il >> 3) * 4096 + (il & 7) * 128
                                         + jpart, LANES)]
                sbase = (u >> 5) * blk_w + jpart
                for h in range(HEADS):
                    val = _vreg_gather(w_cols[h], idx_vec)
                    off = (h // 8) * 4096 + (h % 8) * 128
                    row_bufs[rb][pl.ds(sbase + off, LANES)] = val

        def issue_out(c, rb):
            return pltpu.async_copy(
                row_bufs[rb],
                out_hbm.at[pl.ds(out_base + c * chunk_w, chunk_w)],
                out_sems[rb])

        def wait_out(rb):
            pltpu.make_async_copy(
                row_bufs[rb],
                out_hbm.at[pl.ds(0, chunk_w)],
                out_sems[rb]).wait()

        compute_chunk(0, 0)
        issue_out(0, 0)
        compute_chunk(1, 1)
        issue_out(1, 1)

        def pair_body(t, carry):
            c0 = t * 2

            @pl.when(t == n_pairs // 2)
            def _():
                pltpu.make_async_copy(
                    x_hbm.at[pl.ds(0, half_w)],
                    idx_vmem.at[pl.ds(half_w, half_w)], idx_sem2).wait()

            wait_out(0)
            compute_chunk(c0, 0)
            issue_out(c0, 0)
            wait_out(1)
            compute_chunk(c0 + 1, 1)
            issue_out(c0 + 1, 1)
            return carry

        lax.fori_loop(1, n_pairs, pair_body, 0, unroll=False)

        wait_out(0)
        wait_out(1)

    return sc_kernel


def kernel(x, spatial_encoding_weight):
    rows, cols = x.shape
    x_flat = (x.reshape(rows // 8, 8, cols // 128, 128)
              .transpose(0, 2, 1, 3).reshape(-1))
    flat = _build_sc_kernel(rows, cols)(
        x_flat, spatial_encoding_weight.reshape(-1))
    f5 = flat.reshape(rows, HEADS // 8, cols // 128, 8, 128)
    return f5.transpose(0, 2, 4, 1, 3).reshape(rows, cols, HEADS)
